# baseline (speedup 1.0000x reference)
import jax
import jax.numpy as jnp
from jax.experimental import pallas as pl
from jax.experimental.pallas import tpu as pltpu

_BB = 32
_NBUF = 6


def _blend_body(y_ref, t_ref, x_hbm, o_hbm, ibuf, obuf, isem, osem):
    i = pl.program_id(0)
    B = x_hbm.shape[0]
    n = B // _BB
    R, S = ibuf.shape[-2], ibuf.shape[-1]
    x3 = x_hbm.reshape(B, R, S)
    o3 = o_hbm.reshape(B, R, S)
    slot = jax.lax.rem(i, _NBUF)

    def in_copy(step, buf):
        return pltpu.make_async_copy(
            x3.at[pl.ds(step * _BB, _BB)], ibuf.at[buf], isem.at[buf])

    def out_copy(step, buf):
        return pltpu.make_async_copy(
            obuf.at[buf], o3.at[pl.ds(step * _BB, _BB)], osem.at[buf])

    @pl.when(i == 0)
    def _prologue():
        for k in range(min(_NBUF, n)):
            in_copy(k, k).start()

    in_copy(i, slot).wait()

    @pl.when(i >= _NBUF)
    def _wait_prev_out():
        out_copy(i - _NBUF, slot).wait()

    m = (y_ref[...] == 0).astype(jnp.float32)
    t = t_ref[...]
    xv = ibuf[slot]
    obuf[slot] = xv - (m[:, :, None] * t) * (xv + 0.75)

    out_copy(i, slot).start()

    @pl.when(i + _NBUF < n)
    def _next_in():
        in_copy(i + _NBUF, slot).start()

    @pl.when(i == n - 1)
    def _epilogue():
        for k in range(max(n - _NBUF, 0), n):
            out_copy(k, k % _NBUF).wait()


def kernel(x, y, template):
    B, C, S, _ = x.shape
    R = C * S
    t3 = jnp.tile(template, (C, 1)).reshape(1, R, S)
    out = pl.pallas_call(
        _blend_body,
        grid=(B // _BB,),
        in_specs=[
            pl.BlockSpec((_BB, 1), lambda i: (i, 0)),
            pl.BlockSpec((1, R, S), lambda i: (0, 0, 0)),
            pl.BlockSpec(memory_space=pltpu.MemorySpace.HBM),
        ],
        out_specs=pl.BlockSpec(memory_space=pltpu.MemorySpace.HBM),
        out_shape=jax.ShapeDtypeStruct(x.shape, x.dtype),
        scratch_shapes=[
            pltpu.VMEM((_NBUF, _BB, R, S), jnp.float32),
            pltpu.VMEM((_NBUF, _BB, R, S), jnp.float32),
            pltpu.SemaphoreType.DMA((_NBUF,)),
            pltpu.SemaphoreType.DMA((_NBUF,)),
        ],
    )(y, t3, x)
    return (out, y)


# speedup vs baseline: 6.3797x; 6.3797x over previous
import jax
import jax.numpy as jnp
from jax.experimental import pallas as pl

_FB = 384


def _blend_body(y_ref, t_ref, x_ref, o_ref):
    m = (y_ref[...] == 0).astype(jnp.float32)
    t = t_ref[...]
    xv = x_ref[...]
    o_ref[...] = xv - ((xv + 0.75) * m) * t


def kernel(x, y, template):
    B, C, S, _ = x.shape
    F = C * S * S
    xt = x.transpose(1, 2, 3, 0).reshape(F, B)
    yt = y.reshape(1, B)
    tcol = jnp.tile(template.reshape(-1), C).reshape(F, 1)
    out = pl.pallas_call(
        _blend_body,
        grid=(F // _FB,),
        in_specs=[
            pl.BlockSpec((1, B), lambda i: (0, 0)),
            pl.BlockSpec((_FB, 1), lambda i: (i, 0)),
            pl.BlockSpec((_FB, B), lambda i: (i, 0)),
        ],
        out_specs=pl.BlockSpec((_FB, B), lambda i: (i, 0)),
        out_shape=jax.ShapeDtypeStruct((F, B), x.dtype),
    )(yt, tcol, xt)
    return (out.reshape(C, S, S, B).transpose(3, 0, 1, 2), y)
